# baseline (device time: 104355 ns/iter reference)
import jax
import jax.numpy as jnp
from jax import lax
from jax.experimental import pallas as pl
from jax.experimental.pallas import tpu as pltpu

N_DEV = 8
B_LOC = 2
H_LOC = 4
SQ = 128
DH = 64
D_MODEL = 512
D_CHUNK = H_LOC * DH


def kernel(x, Wq, K_ext, V_ext, Wo):
    my = lax.axis_index("i")
    k_loc = lax.dynamic_slice_in_dim(K_ext, my * B_LOC, B_LOC, axis=0)
    v_loc = lax.dynamic_slice_in_dim(V_ext, my * B_LOC, B_LOC, axis=0)
    k_t = jnp.transpose(k_loc, (0, 2, 1, 3))
    v_t = jnp.transpose(v_loc, (0, 2, 1, 3))

    def body(x_ref, wq_ref, k_ref, v_ref, wo_ref, out_ref,
             wq_comm, wo_comm, q_send, q_recv, o_send, o_recv):
        my_pos = lax.axis_index("i")
        left = jnp.mod(my_pos - 1, N_DEV)
        right = jnp.mod(my_pos + 1, N_DEV)

        barrier_sem = pltpu.get_barrier_semaphore()
        for nbr in (left, right):
            pl.semaphore_signal(
                barrier_sem, inc=1,
                device_id=(nbr,), device_id_type=pl.DeviceIdType.MESH,
            )
        pl.semaphore_wait(barrier_sem, 2)

        wq_comm[0] = wq_ref[...]
        wo_comm[0] = wo_ref[...]

        qb = lax.broadcasted_iota(jnp.int32, (SQ, SQ), 0) // 64
        kb = lax.broadcasted_iota(jnp.int32, (SQ, SQ), 1) // 64
        mask = (qb == kb) | ((kb % 4) == (qb % 4))

        def compute_chunk(s):
            src = jnp.mod(my_pos - s, N_DEV)
            wq_c = wq_comm[s]
            wo_c = wo_comm[s]
            for b in range(B_LOC):
                q_full = jnp.dot(x_ref[b], wq_c,
                                 preferred_element_type=jnp.float32)
                ctx_parts = []
                for h in range(H_LOC):
                    hg = src * H_LOC + h
                    q = q_full[:, h * DH:(h + 1) * DH]
                    k = k_ref[b, hg]
                    v = v_ref[b, hg]
                    sc = lax.dot_general(
                        q, k, (((1,), (1,)), ((), ())),
                        preferred_element_type=jnp.float32) * 0.125
                    sc = jnp.where(mask, sc, -1e9)
                    m = jnp.max(sc, axis=-1, keepdims=True)
                    w = jnp.exp(sc - m)
                    w = w / jnp.sum(w, axis=-1, keepdims=True)
                    ctx_parts.append(
                        jnp.dot(w, v, preferred_element_type=jnp.float32))
                ctx = jnp.concatenate(ctx_parts, axis=-1)
                contrib = jnp.dot(ctx, wo_c,
                                  preferred_element_type=jnp.float32)
                if s == 0:
                    out_ref[b] = contrib
                else:
                    out_ref[b] = out_ref[b] + contrib

        for j in range(1, N_DEV):
            rq = pltpu.make_async_remote_copy(
                src_ref=wq_comm.at[j - 1], dst_ref=wq_comm.at[j],
                send_sem=q_send.at[j - 1], recv_sem=q_recv.at[j - 1],
                device_id=(right,), device_id_type=pl.DeviceIdType.MESH,
            )
            ro = pltpu.make_async_remote_copy(
                src_ref=wo_comm.at[j - 1], dst_ref=wo_comm.at[j],
                send_sem=o_send.at[j - 1], recv_sem=o_recv.at[j - 1],
                device_id=(right,), device_id_type=pl.DeviceIdType.MESH,
            )
            rq.start()
            ro.start()
            compute_chunk(j - 1)
            rq.wait()
            ro.wait()
        compute_chunk(N_DEV - 1)

    out_shape = jax.ShapeDtypeStruct((B_LOC, SQ, D_MODEL), jnp.float32)
    return pl.pallas_call(
        body,
        out_shape=out_shape,
        in_specs=[pl.BlockSpec(memory_space=pltpu.VMEM)] * 5,
        out_specs=pl.BlockSpec(memory_space=pltpu.VMEM),
        scratch_shapes=[
            pltpu.VMEM((N_DEV, D_MODEL, D_CHUNK), jnp.float32),
            pltpu.VMEM((N_DEV, D_CHUNK, D_MODEL), jnp.float32),
            pltpu.SemaphoreType.DMA((N_DEV - 1,)),
            pltpu.SemaphoreType.DMA((N_DEV - 1,)),
            pltpu.SemaphoreType.DMA((N_DEV - 1,)),
            pltpu.SemaphoreType.DMA((N_DEV - 1,)),
        ],
        compiler_params=pltpu.CompilerParams(collective_id=0),
    )(x, Wq, k_t, v_t, Wo)


# device time: 47408 ns/iter; 2.2012x vs baseline; 2.2012x over previous
import jax
import jax.numpy as jnp
from jax import lax
from jax.experimental import pallas as pl
from jax.experimental.pallas import tpu as pltpu

N_DEV = 8
FWD = 4
BWD = 3
B_LOC = 2
H_LOC = 4
SQ = 128
DH = 64
D_MODEL = 512
D_CHUNK = H_LOC * DH


def kernel(x, Wq, K_ext, V_ext, Wo):
    my = lax.axis_index("i")
    k_loc = lax.dynamic_slice_in_dim(K_ext, my * B_LOC, B_LOC, axis=0)
    v_loc = lax.dynamic_slice_in_dim(V_ext, my * B_LOC, B_LOC, axis=0)
    k_t = jnp.transpose(k_loc, (0, 2, 1, 3)).astype(jnp.bfloat16)
    v_t = jnp.transpose(v_loc, (0, 2, 1, 3)).astype(jnp.bfloat16)
    x_b = x.astype(jnp.bfloat16)
    chunk = jnp.concatenate(
        [Wq.astype(jnp.bfloat16), Wo.T.astype(jnp.bfloat16)], axis=0)

    def body(x_ref, chunk_ref, k_ref, v_ref, out_ref,
             comm, send_f, recv_f, send_b, recv_b):
        my_pos = lax.axis_index("i")
        left = jnp.mod(my_pos - 1, N_DEV)
        right = jnp.mod(my_pos + 1, N_DEV)

        barrier_sem = pltpu.get_barrier_semaphore()
        for nbr in (left, right):
            pl.semaphore_signal(
                barrier_sem, inc=1,
                device_id=(nbr,), device_id_type=pl.DeviceIdType.MESH,
            )
        pl.semaphore_wait(barrier_sem, 2)

        comm[0] = chunk_ref[...]

        qb = lax.broadcasted_iota(jnp.int32, (SQ, SQ), 0) // 64
        kb = lax.broadcasted_iota(jnp.int32, (SQ, SQ), 1) // 64
        mask = (qb == kb) | ((kb % 4) == (qb % 4))

        def compute_chunk(slot, origin, first=False):
            src = jnp.mod(origin, N_DEV)
            wq_c = comm[slot, :D_MODEL, :]
            woT_c = comm[slot, D_MODEL:, :]
            for b in range(B_LOC):
                q_full = jnp.dot(x_ref[b], wq_c,
                                 preferred_element_type=jnp.float32)
                ctx_parts = []
                for h in range(H_LOC):
                    hg = src * H_LOC + h
                    q = q_full[:, h * DH:(h + 1) * DH].astype(jnp.bfloat16)
                    k = k_ref[b, hg]
                    v = v_ref[b, hg]
                    sc = lax.dot_general(
                        q, k, (((1,), (1,)), ((), ())),
                        preferred_element_type=jnp.float32) * 0.125
                    sc = jnp.where(mask, sc, -1e9)
                    m = jnp.max(sc, axis=-1, keepdims=True)
                    w = jnp.exp(sc - m)
                    w = (w / jnp.sum(w, axis=-1, keepdims=True)
                         ).astype(jnp.bfloat16)
                    ctx_parts.append(
                        jnp.dot(w, v, preferred_element_type=jnp.float32))
                ctx = jnp.concatenate(ctx_parts, axis=-1).astype(jnp.bfloat16)
                contrib = lax.dot_general(
                    ctx, woT_c, (((1,), (1,)), ((), ())),
                    preferred_element_type=jnp.float32)
                if first:
                    out_ref[b] = contrib
                else:
                    out_ref[b] = out_ref[b] + contrib

        for r in range(1, FWD + 1):
            rf = pltpu.make_async_remote_copy(
                src_ref=comm.at[r - 1], dst_ref=comm.at[r],
                send_sem=send_f.at[r - 1], recv_sem=recv_f.at[r - 1],
                device_id=(right,), device_id_type=pl.DeviceIdType.MESH,
            )
            rf.start()
            if r <= BWD:
                rb = pltpu.make_async_remote_copy(
                    src_ref=comm.at[0 if r == 1 else 4 + (r - 1)],
                    dst_ref=comm.at[4 + r],
                    send_sem=send_b.at[r - 1], recv_sem=recv_b.at[r - 1],
                    device_id=(left,), device_id_type=pl.DeviceIdType.MESH,
                )
                rb.start()
            if r == 1:
                compute_chunk(0, my_pos, first=True)
            else:
                compute_chunk(r - 1, my_pos - (r - 1))
                compute_chunk(4 + (r - 1), my_pos + (r - 1))
            rf.wait()
            if r <= BWD:
                rb.wait()
        compute_chunk(FWD, my_pos - FWD)

    out_shape = jax.ShapeDtypeStruct((B_LOC, SQ, D_MODEL), jnp.float32)
    return pl.pallas_call(
        body,
        out_shape=out_shape,
        in_specs=[pl.BlockSpec(memory_space=pltpu.VMEM)] * 4,
        out_specs=pl.BlockSpec(memory_space=pltpu.VMEM),
        scratch_shapes=[
            pltpu.VMEM((N_DEV, 2 * D_MODEL, D_CHUNK), jnp.bfloat16),
            pltpu.SemaphoreType.DMA((FWD,)),
            pltpu.SemaphoreType.DMA((FWD,)),
            pltpu.SemaphoreType.DMA((BWD,)),
            pltpu.SemaphoreType.DMA((BWD,)),
        ],
        compiler_params=pltpu.CompilerParams(collective_id=0),
    )(x_b, chunk, k_t, v_t)
